# baseline (speedup 1.0000x reference)
import jax
import jax.numpy as jnp
from jax import lax
from jax.experimental import pallas as pl
from jax.experimental.pallas import tpu as pltpu
from jax.experimental.pallas import tpu_sc as plsc

NN = 10000
DIN = 128
DOUT = 64
D1 = 96
NC = 2
NS = 16
NW = NC * NS
CHUNK = 1024
K = 10
RB = 128
E_PAD = NW * K * CHUNK
ROWS_PER_TILE = 640
ROWS = NS * ROWS_PER_TILE
DUMP_ROW = NN

_MESH = plsc.VectorSubcoreMesh(
    core_axis_name="c", subcore_axis_name="s", num_cores=NC, num_subcores=NS)


def _make_sc_agg(D):
    def body(tab, srcb, dstb, out_acc, src_v, dst_v, rows_v, stage_v, zbuf,
             acc_sh, sem):
        c = lax.axis_index("c")
        s = lax.axis_index("s")
        w = c * NS + s
        pltpu.sync_copy(srcb.at[w], src_v)
        pltpu.sync_copy(dstb.at[w], dst_v)
        zeros32 = jnp.zeros((32,), jnp.bfloat16)
        for r in range(16):
            for t in range(D // 32):
                zbuf[r, pl.ds(t * 32, 32)] = zeros32
        base = s * ROWS_PER_TILE

        def zacc(i, carry):
            pltpu.sync_copy(zbuf, acc_sh.at[pl.ds(base + i * 16, 16)])
            return carry

        with jax.named_scope("agg_zero"):
            lax.fori_loop(0, ROWS_PER_TILE // 16, zacc, 0)
            plsc.subcore_barrier()

        def step(j, carry):
            with jax.named_scope("agg_gather"):
                pltpu.async_copy(tab.at[src_v.at[j]], rows_v, sem).wait()
            with jax.named_scope("agg_scatter"):
                pltpu.sync_copy(rows_v, acc_sh.at[dst_v.at[j]], add=True)
            return carry

        with jax.named_scope("agg_main"):
            lax.fori_loop(0, K, step, 0)
            plsc.subcore_barrier()

        def wout(i, carry):
            off = base + i * RB
            pltpu.sync_copy(acc_sh.at[pl.ds(off, RB)], stage_v)
            pltpu.sync_copy(stage_v, out_acc.at[c, pl.ds(off, RB)])
            return carry

        with jax.named_scope("agg_out"):
            lax.fori_loop(0, ROWS_PER_TILE // RB, wout, 0)

    return pl.kernel(
        body,
        out_type=jax.ShapeDtypeStruct((NC, ROWS, D), jnp.bfloat16),
        mesh=_MESH,
        scratch_types=(
            pltpu.VMEM((K, CHUNK), jnp.int32),
            pltpu.VMEM((K, CHUNK), jnp.int32),
            pltpu.VMEM((CHUNK, D), jnp.bfloat16),
            pltpu.VMEM((RB, D), jnp.bfloat16),
            pltpu.VMEM((16, D), jnp.bfloat16),
            pltpu.VMEM_SHARED((ROWS, D), jnp.bfloat16),
            pltpu.SemaphoreType.DMA,
        ),
        compiler_params=pltpu.CompilerParams(use_tc_tiling_on_sc=False),
    )


_sc_agg1 = _make_sc_agg(D1)
_sc_agg2 = _make_sc_agg(DOUT)


def _dot_t(a, b):
    return lax.dot_general(a, b, (((1,), (1,)), ((), ())),
                           preferred_element_type=jnp.float32)


def _tc1_body(x_ref, wl_ref, wr_ref, b_ref, tab_ref, s_ref):
    xv = x_ref[...]
    xw = _dot_t(xv, wl_ref[...])
    cols = lax.broadcasted_iota(jnp.int32, (NN, D1 - DOUT), 1)
    tail = jnp.where(cols == 0, jnp.float32(1.0), jnp.float32(0.0))
    tab_ref[...] = jnp.concatenate([xw, tail], axis=1).astype(jnp.bfloat16)
    s_ref[...] = _dot_t(xv, wr_ref[...]) + b_ref[...][None, :]


_tc1 = pl.pallas_call(
    _tc1_body,
    out_shape=(jax.ShapeDtypeStruct((NN, D1), jnp.bfloat16),
               jax.ShapeDtypeStruct((NN, DOUT), jnp.float32)))


def _tc2_body(acc_ref, s1_ref, wl_ref, wr_ref, b_ref, tab2_ref, s2_ref):
    p = acc_ref[0].astype(jnp.float32) + acc_ref[1].astype(jnp.float32)
    feat = p[:NN, :DOUT]
    cnt = p[:NN, DOUT:DOUT + 1]
    inv = 1.0 / jnp.clip(cnt, 1.0, None)
    h = feat * inv + s1_ref[...]
    tab2_ref[...] = _dot_t(h, wl_ref[...]).astype(jnp.bfloat16)
    s2_ref[...] = _dot_t(h, wr_ref[...]) + b_ref[...][None, :]


_tc2 = pl.pallas_call(
    _tc2_body,
    out_shape=(jax.ShapeDtypeStruct((NN, DOUT), jnp.bfloat16),
               jax.ShapeDtypeStruct((NN, DOUT), jnp.float32)))


def _tc3_body(acc2_ref, acc1_ref, s2_ref, out_ref):
    p2 = acc2_ref[0].astype(jnp.float32) + acc2_ref[1].astype(jnp.float32)
    cnt = (acc1_ref[0, :NN, DOUT:DOUT + 1].astype(jnp.float32)
           + acc1_ref[1, :NN, DOUT:DOUT + 1].astype(jnp.float32))
    inv = 1.0 / jnp.clip(cnt, 1.0, None)
    out_ref[...] = p2[:NN] * inv + s2_ref[...]


_tc3 = pl.pallas_call(
    _tc3_body,
    out_shape=jax.ShapeDtypeStruct((NN, DOUT), jnp.float32))


def kernel(x, edge_index, W1_l, b1_l, W1_r, W2_l, b2_l, W2_r):
    src = edge_index[0].astype(jnp.int32)
    dst = edge_index[1].astype(jnp.int32)
    pad = E_PAD - src.shape[0]
    srcb = jnp.concatenate([src, jnp.zeros((pad,), jnp.int32)]).reshape(NW, K, CHUNK)
    dstb = jnp.concatenate([dst, jnp.full((pad,), DUMP_ROW, jnp.int32)]).reshape(NW, K, CHUNK)

    tab1, s1 = _tc1(x, W1_l, W1_r, b1_l)
    acc1 = _sc_agg1(tab1, srcb, dstb)
    tab2, s2 = _tc2(acc1, s1, W2_l, W2_r, b2_l)
    acc2 = _sc_agg2(tab2, srcb, dstb)
    return _tc3(acc2, acc1, s2)


# speedup vs baseline: 1.0029x; 1.0029x over previous
import jax
import jax.numpy as jnp
from jax import lax
from jax.experimental import pallas as pl
from jax.experimental.pallas import tpu as pltpu
from jax.experimental.pallas import tpu_sc as plsc

NN = 10000
DIN = 128
DOUT = 64
D1 = 96
NC = 2
NS = 16
NW = NC * NS
CHUNK = 1024
K = 10
RB = 128
E_PAD = NW * K * CHUNK
ROWS_PER_TILE = 640
ROWS = NS * ROWS_PER_TILE
DUMP_ROW = NN

_MESH = plsc.VectorSubcoreMesh(
    core_axis_name="c", subcore_axis_name="s", num_cores=NC, num_subcores=NS)


def _make_sc_agg(D):
    def body(tab, srcb, dstb, out_acc, src_v, dst_v, rows_v, stage_v, zbuf,
             acc_sh, sem):
        c = lax.axis_index("c")
        s = lax.axis_index("s")
        w = c * NS + s
        pltpu.sync_copy(srcb.at[w], src_v)
        pltpu.sync_copy(dstb.at[w], dst_v)
        zeros32 = jnp.zeros((32,), jnp.bfloat16)
        for r in range(16):
            for t in range(D // 32):
                zbuf[r, pl.ds(t * 32, 32)] = zeros32
        base = s * ROWS_PER_TILE

        def zacc(i, carry):
            pltpu.sync_copy(zbuf, acc_sh.at[pl.ds(base + i * 16, 16)])
            return carry

        with jax.named_scope("agg_zero"):
            lax.fori_loop(0, ROWS_PER_TILE // 16, zacc, 0)
            plsc.subcore_barrier()

        def step(j, carry):
            with jax.named_scope("agg_gather"):
                pltpu.async_copy(tab.at[src_v.at[j]], rows_v, sem).wait()
            with jax.named_scope("agg_scatter"):
                pltpu.sync_copy(rows_v, acc_sh.at[dst_v.at[j]], add=True)
            return carry

        with jax.named_scope("agg_main"):
            lax.fori_loop(0, K, step, 0)
        with jax.named_scope("agg_bar"):
            plsc.subcore_barrier()

        def wout(i, carry):
            off = base + i * RB
            pltpu.sync_copy(acc_sh.at[pl.ds(off, RB)], stage_v)
            pltpu.sync_copy(stage_v, out_acc.at[c, pl.ds(off, RB)])
            return carry

        with jax.named_scope("agg_out"):
            lax.fori_loop(0, ROWS_PER_TILE // RB, wout, 0)

    return pl.kernel(
        body,
        out_type=jax.ShapeDtypeStruct((NC, ROWS, D), jnp.bfloat16),
        mesh=_MESH,
        scratch_types=(
            pltpu.VMEM((K, CHUNK), jnp.int32),
            pltpu.VMEM((K, CHUNK), jnp.int32),
            pltpu.VMEM((CHUNK, D), jnp.bfloat16),
            pltpu.VMEM((RB, D), jnp.bfloat16),
            pltpu.VMEM((16, D), jnp.bfloat16),
            pltpu.VMEM_SHARED((ROWS, D), jnp.bfloat16),
            pltpu.SemaphoreType.DMA,
        ),
        compiler_params=pltpu.CompilerParams(use_tc_tiling_on_sc=False),
    )


_sc_agg1 = _make_sc_agg(D1)
_sc_agg2 = _make_sc_agg(DOUT)


def _dot_t(a, b):
    return lax.dot_general(a, b, (((1,), (1,)), ((), ())),
                           preferred_element_type=jnp.float32)


def _tc1_body(x_ref, wl_ref, wr_ref, b_ref, tab_ref, s_ref):
    xv = x_ref[...]
    xw = _dot_t(xv, wl_ref[...])
    cols = lax.broadcasted_iota(jnp.int32, (NN, D1 - DOUT), 1)
    tail = jnp.where(cols == 0, jnp.float32(1.0), jnp.float32(0.0))
    tab_ref[...] = jnp.concatenate([xw, tail], axis=1).astype(jnp.bfloat16)
    s_ref[...] = _dot_t(xv, wr_ref[...]) + b_ref[...][None, :]


_tc1 = pl.pallas_call(
    _tc1_body,
    out_shape=(jax.ShapeDtypeStruct((NN, D1), jnp.bfloat16),
               jax.ShapeDtypeStruct((NN, DOUT), jnp.float32)))


def _tc2_body(acc_ref, s1_ref, wl_ref, wr_ref, b_ref, tab2_ref, s2_ref):
    p = acc_ref[0].astype(jnp.float32) + acc_ref[1].astype(jnp.float32)
    feat = p[:NN, :DOUT]
    cnt = p[:NN, DOUT:DOUT + 1]
    inv = 1.0 / jnp.clip(cnt, 1.0, None)
    h = feat * inv + s1_ref[...]
    tab2_ref[...] = _dot_t(h, wl_ref[...]).astype(jnp.bfloat16)
    s2_ref[...] = _dot_t(h, wr_ref[...]) + b_ref[...][None, :]


_tc2 = pl.pallas_call(
    _tc2_body,
    out_shape=(jax.ShapeDtypeStruct((NN, DOUT), jnp.bfloat16),
               jax.ShapeDtypeStruct((NN, DOUT), jnp.float32)))


def _tc3_body(acc2_ref, acc1_ref, s2_ref, out_ref):
    p2 = acc2_ref[0].astype(jnp.float32) + acc2_ref[1].astype(jnp.float32)
    cnt = (acc1_ref[0, :NN, DOUT:DOUT + 1].astype(jnp.float32)
           + acc1_ref[1, :NN, DOUT:DOUT + 1].astype(jnp.float32))
    inv = 1.0 / jnp.clip(cnt, 1.0, None)
    out_ref[...] = p2[:NN] * inv + s2_ref[...]


_tc3 = pl.pallas_call(
    _tc3_body,
    out_shape=jax.ShapeDtypeStruct((NN, DOUT), jnp.float32))


def kernel(x, edge_index, W1_l, b1_l, W1_r, W2_l, b2_l, W2_r):
    src = edge_index[0].astype(jnp.int32)
    dst = edge_index[1].astype(jnp.int32)
    pad = E_PAD - src.shape[0]
    srcb = jnp.concatenate([src, jnp.zeros((pad,), jnp.int32)]).reshape(NW, K, CHUNK)
    dstb = jnp.concatenate([dst, jnp.full((pad,), DUMP_ROW, jnp.int32)]).reshape(NW, K, CHUNK)

    tab1, s1 = _tc1(x, W1_l, W1_r, b1_l)
    acc1 = _sc_agg1(tab1, srcb, dstb)
    tab2, s2 = _tc2(acc1, s1, W2_l, W2_r, b2_l)
    acc2 = _sc_agg2(tab2, srcb, dstb)
    return _tc3(acc2, acc1, s2)


# speedup vs baseline: 1.0986x; 1.0954x over previous
import jax
import jax.numpy as jnp
from jax import lax
from jax.experimental import pallas as pl
from jax.experimental.pallas import tpu as pltpu
from jax.experimental.pallas import tpu_sc as plsc

NN = 10000
DIN = 128
DOUT = 64
D1 = 96
NC = 2
NS = 16
NW = NC * NS
CHUNK = 1024
K0 = 15
K1 = 5
RB = 128
E_PAD = NS * (K0 + K1) * CHUNK
ROWS_PER_TILE = 640
ROWS = NS * ROWS_PER_TILE
DUMP_ROW = NN

_MESH = plsc.VectorSubcoreMesh(
    core_axis_name="c", subcore_axis_name="s", num_cores=NC, num_subcores=NS)


def _make_sc_agg(D):
    def body(tab, srcb, dstb, out_acc, src_v, dst_v, rows_v, stage_v, zbuf,
             acc_sh, sem):
        c = lax.axis_index("c")
        s = lax.axis_index("s")
        w = c * NS + s
        pltpu.sync_copy(srcb.at[w], src_v)
        pltpu.sync_copy(dstb.at[w], dst_v)
        zeros32 = jnp.zeros((32,), jnp.bfloat16)
        for r in range(16):
            for t in range(D // 32):
                zbuf[r, pl.ds(t * 32, 32)] = zeros32
        base = s * ROWS_PER_TILE

        def zacc(i, carry):
            pltpu.sync_copy(zbuf, acc_sh.at[pl.ds(base + i * 16, 16)])
            return carry

        with jax.named_scope("agg_zero"):
            lax.fori_loop(0, ROWS_PER_TILE // 16, zacc, 0)
            plsc.subcore_barrier()

        def step(j, carry):
            with jax.named_scope("agg_gather"):
                pltpu.async_copy(tab.at[src_v.at[j]], rows_v, sem).wait()
            with jax.named_scope("agg_scatter"):
                pltpu.sync_copy(rows_v, acc_sh.at[dst_v.at[j]], add=True)
            return carry

        with jax.named_scope("agg_main"):
            nblk = jnp.where(c == 0, K0, K1)
            lax.fori_loop(0, nblk, step, 0)
        with jax.named_scope("agg_bar"):
            plsc.subcore_barrier()

        def wout(i, carry):
            off = base + i * RB
            pltpu.sync_copy(acc_sh.at[pl.ds(off, RB)], stage_v)
            pltpu.sync_copy(stage_v, out_acc.at[c, pl.ds(off, RB)])
            return carry

        with jax.named_scope("agg_out"):
            lax.fori_loop(0, ROWS_PER_TILE // RB, wout, 0)

    return pl.kernel(
        body,
        out_type=jax.ShapeDtypeStruct((NC, ROWS, D), jnp.bfloat16),
        mesh=_MESH,
        scratch_types=(
            pltpu.VMEM((K0, CHUNK), jnp.int32),
            pltpu.VMEM((K0, CHUNK), jnp.int32),
            pltpu.VMEM((CHUNK, D), jnp.bfloat16),
            pltpu.VMEM((RB, D), jnp.bfloat16),
            pltpu.VMEM((16, D), jnp.bfloat16),
            pltpu.VMEM_SHARED((ROWS, D), jnp.bfloat16),
            pltpu.SemaphoreType.DMA,
        ),
        compiler_params=pltpu.CompilerParams(use_tc_tiling_on_sc=False),
    )


_sc_agg1 = _make_sc_agg(D1)
_sc_agg2 = _make_sc_agg(DOUT)


def _dot_t(a, b):
    return lax.dot_general(a, b, (((1,), (1,)), ((), ())),
                           preferred_element_type=jnp.float32)


def _tc1_body(x_ref, wl_ref, wr_ref, b_ref, tab_ref, s_ref):
    xv = x_ref[...]
    xw = _dot_t(xv, wl_ref[...])
    cols = lax.broadcasted_iota(jnp.int32, (NN, D1 - DOUT), 1)
    tail = jnp.where(cols == 0, jnp.float32(1.0), jnp.float32(0.0))
    tab_ref[...] = jnp.concatenate([xw, tail], axis=1).astype(jnp.bfloat16)
    s_ref[...] = _dot_t(xv, wr_ref[...]) + b_ref[...][None, :]


_tc1 = pl.pallas_call(
    _tc1_body,
    out_shape=(jax.ShapeDtypeStruct((NN, D1), jnp.bfloat16),
               jax.ShapeDtypeStruct((NN, DOUT), jnp.float32)))


def _tc2_body(acc_ref, s1_ref, wl_ref, wr_ref, b_ref, tab2_ref, s2_ref):
    p = acc_ref[0].astype(jnp.float32) + acc_ref[1].astype(jnp.float32)
    feat = p[:NN, :DOUT]
    cnt = p[:NN, DOUT:DOUT + 1]
    inv = 1.0 / jnp.clip(cnt, 1.0, None)
    h = feat * inv + s1_ref[...]
    tab2_ref[...] = _dot_t(h, wl_ref[...]).astype(jnp.bfloat16)
    s2_ref[...] = _dot_t(h, wr_ref[...]) + b_ref[...][None, :]


_tc2 = pl.pallas_call(
    _tc2_body,
    out_shape=(jax.ShapeDtypeStruct((NN, DOUT), jnp.bfloat16),
               jax.ShapeDtypeStruct((NN, DOUT), jnp.float32)))


def _tc3_body(acc2_ref, acc1_ref, s2_ref, out_ref):
    p2 = acc2_ref[0].astype(jnp.float32) + acc2_ref[1].astype(jnp.float32)
    cnt = (acc1_ref[0, :NN, DOUT:DOUT + 1].astype(jnp.float32)
           + acc1_ref[1, :NN, DOUT:DOUT + 1].astype(jnp.float32))
    inv = 1.0 / jnp.clip(cnt, 1.0, None)
    out_ref[...] = p2[:NN] * inv + s2_ref[...]


_tc3 = pl.pallas_call(
    _tc3_body,
    out_shape=jax.ShapeDtypeStruct((NN, DOUT), jnp.float32))


def kernel(x, edge_index, W1_l, b1_l, W1_r, W2_l, b2_l, W2_r):
    src = edge_index[0].astype(jnp.int32)
    dst = edge_index[1].astype(jnp.int32)

    def blocks(flat, fill):
        pad = E_PAD - flat.shape[0]
        flat = jnp.concatenate([flat, jnp.full((pad,), fill, jnp.int32)])
        n0 = NS * K0 * CHUNK
        p0 = flat[:n0].reshape(NS, K0, CHUNK)
        p1 = flat[n0:].reshape(NS, K1, CHUNK)
        p1 = jnp.concatenate(
            [p1, jnp.full((NS, K0 - K1, CHUNK), fill, jnp.int32)], axis=1)
        return jnp.concatenate([p0, p1], axis=0)

    srcb = blocks(src, 0)
    dstb = blocks(dst, DUMP_ROW)

    tab1, s1 = _tc1(x, W1_l, W1_r, b1_l)
    acc1 = _sc_agg1(tab1, srcb, dstb)
    tab2, s2 = _tc2(acc1, s1, W2_l, W2_r, b2_l)
    acc2 = _sc_agg2(tab2, srcb, dstb)
    return _tc3(acc2, acc1, s2)
